# baseline (speedup 1.0000x reference)
import functools

import jax
import jax.numpy as jnp
from jax import lax
from jax.experimental import pallas as pl
from jax.experimental.pallas import tpu as pltpu
from jax.experimental.pallas import tpu_sc as plsc

N = 100000
D = 128
S = 256
R = 4000
NB = N // R
NEG = -1e30
SW = 64
NST = S // SW

NC, NS, NW = 2, 16, 32
T = 400
SB = 80
NSB = T // SB
NCHUNK = N // T
CPW = (NCHUNK + NW - 1) // NW

_INTERPRET = False


def _onehot(seg, dtype=jnp.float32):
    cols = lax.broadcasted_iota(jnp.int32, (R, S), 1)
    return jnp.where(seg[:, None] == cols, jnp.array(1.0, dtype), jnp.array(0.0, dtype))


def _sc_pass1_body(xa_hbm, sa_hbm, xb_hbm, sb_hbm, sums_out,
                   xbufs, idxbufs, zbuf, semx0, semx1,
                   tab_a, tab_b):
    cid = lax.axis_index("c")
    sid = lax.axis_index("s")
    w = sid * NC + cid
    sems = (semx0, semx1)

    zero = jnp.zeros((16,), jnp.float32)
    for r in range(16):
        for j in range(D // 16):
            zbuf[r, pl.ds(16 * j, 16)] = zero

    row0 = sid * 16
    pltpu.sync_copy(zbuf, tab_a.at[pl.ds(row0, 16)])
    pltpu.sync_copy(zbuf, tab_b.at[pl.ds(row0, 16)])
    plsc.subcore_barrier()

    def _copies(x_hbm, s_hbm, k, b):
        off = k * T
        sem = sems[b]
        cps = [pltpu.make_async_copy(x_hbm.at[pl.ds(off, T)], xbufs.at[b], sem)]
        for i in range(NSB):
            cps.append(pltpu.make_async_copy(
                s_hbm.at[pl.ds(off + i * SB, SB)], idxbufs.at[b, i], sem))
        return cps

    for x_hbm, s_hbm, tab in ((xa_hbm, sa_hbm, tab_a),
                              (xb_hbm, sb_hbm, tab_b)):
        @pl.when(w < NCHUNK)
        def _():
            for cp in _copies(x_hbm, s_hbm, w, 0):
                cp.start()

        for j in range(CPW):
            k = w + NW * j
            b = j % 2
            if j + 1 < CPW:
                k_next = k + NW

                @pl.when(k_next < NCHUNK)
                def _():
                    for cp in _copies(x_hbm, s_hbm, k_next, 1 - b):
                        cp.start()

            @pl.when(k < NCHUNK)
            def _():
                for cp in _copies(x_hbm, s_hbm, k, b):
                    cp.wait()
                for i in range(NSB):
                    pltpu.sync_copy(xbufs.at[b, pl.ds(i * SB, SB)],
                                    tab.at[idxbufs.at[b, i]], add=True)

    plsc.subcore_barrier()

    @pl.when(sid == 0)
    def _():
        pltpu.sync_copy(tab_a, sums_out.at[0, cid])
        pltpu.sync_copy(tab_b, sums_out.at[1, cid])


def _sc_pass1(x_A, segA_i32, x_B, segB_i32):
    return pl.kernel(
        _sc_pass1_body,
        out_type=jax.ShapeDtypeStruct((2, NC, S, D), jnp.float32),
        mesh=plsc.VectorSubcoreMesh(core_axis_name="c", subcore_axis_name="s"),
        scratch_types=[
            pltpu.VMEM((2, T, D), jnp.float32),
            pltpu.VMEM((2, NSB, SB), jnp.int32),
            pltpu.VMEM((16, D), jnp.float32),
            pltpu.SemaphoreType.DMA,
            pltpu.SemaphoreType.DMA,
            pltpu.VMEM_SHARED((S, D), jnp.float32),
            pltpu.VMEM_SHARED((S, D), jnp.float32),
        ],
    )(x_A, segA_i32, x_B, segB_i32)


def _g_body(sa_ref, sb_ref, sums_ref, wg_ref, wa_ref, g_ref):
    wg = wg_ref[...]
    wa = wa_ref[...]
    m1 = jnp.dot(wg, wa, preferred_element_type=jnp.float32)
    M = jnp.dot(m1, wg.T, preferred_element_type=jnp.float32)
    cnts = []
    for s_ref in (sa_ref, sb_ref):
        c = jnp.zeros((S,), jnp.float32)
        for j in range(NB):
            c += jnp.sum(_onehot(s_ref[j, 0, :]), axis=0)
        cnts.append(c)
    sums = sums_ref[...]
    tot = sums[:, 0] + sums[:, 1]
    cnt = jnp.maximum(jnp.stack(cnts), 1.0)
    mean = tot / cnt[:, :, None]
    g_ref[0] = jnp.dot(mean[1], M, preferred_element_type=jnp.float32)
    g_ref[1] = jnp.dot(mean[0], M, preferred_element_type=jnp.float32)


def _pass2_body(smin_ref, smax_ref, xa_ref, sa_ref, xb_ref, sb_ref, g_ref,
                u_ref, m_ref, d_ref, g_buf):
    i = pl.program_id(0)

    @pl.when(i == 0)
    def _():
        u_ref[...] = jnp.zeros_like(u_ref)
        m_ref[...] = jnp.full_like(m_ref, NEG)
        d_ref[...] = jnp.zeros_like(d_ref)

    for side, (x_ref, s_ref) in enumerate(((xa_ref, sa_ref), (xb_ref, sb_ref))):
        seg = s_ref[0, 0, :]
        smin = smin_ref[side, i]
        smax = smax_ref[side, i]
        ohb = seg[:, None] == lax.broadcasted_iota(jnp.int32, (R, S), 1)
        oh = jnp.where(ohb, 1.0, 0.0)
        x = x_ref[...]

        g_buf[...] = jnp.zeros_like(g_buf)
        for st in range(NST):
            @pl.when((smin < (st + 1) * SW) & (smax >= st * SW))
            def _():
                g_buf[...] += lax.dot_general(
                    oh[:, st * SW:(st + 1) * SW],
                    g_ref[side, pl.ds(st * SW, SW), :],
                    (((1,), (0,)), ((), ())),
                    preferred_element_type=jnp.float32)

        e = jnp.sum(x * g_buf[...], axis=1)
        me = jnp.max(jnp.where(ohb, e[:, None], NEG), axis=0)
        m_old = m_ref[side]
        m_new = jnp.maximum(m_old, me)
        m_gath = lax.dot_general(oh, m_new[:, None], (((1,), (0,)), ((), ())),
                                 preferred_element_type=jnp.float32)
        w = jnp.exp(e[:, None] - m_gath)
        scale = jnp.exp(m_old - m_new)
        dd = lax.dot_general(oh, w, (((0,), (0,)), ((), ())),
                             preferred_element_type=jnp.float32)
        d_ref[side] = d_ref[side] * scale + dd[:, 0]
        wx = x * w
        for st in range(NST):
            @pl.when((smin < (st + 1) * SW) & (smax >= st * SW))
            def _():
                sl = pl.ds(st * SW, SW)
                u_ref[side, sl, :] = (
                    u_ref[side, sl, :] * scale[st * SW:(st + 1) * SW][:, None]
                    + lax.dot_general(oh[:, st * SW:(st + 1) * SW], wx,
                                      (((0,), (0,)), ((), ())),
                                      preferred_element_type=jnp.float32))
        m_ref[side] = m_new


def _final_body(u_ref, d_ref, wg_ref, outa_ref, outb_ref):
    wg = wg_ref[...]
    den = d_ref[...] + 1e-16
    pooled = u_ref[...] / den[:, :, None]
    outa_ref[...] = jnp.dot(pooled[0], wg, preferred_element_type=jnp.float32)
    outb_ref[...] = jnp.dot(pooled[1], wg, preferred_element_type=jnp.float32)


def kernel(x_A, herb_batch_A, x_B, herb_batch_B, W_gnn, W_attn):
    segA_i32 = herb_batch_A.astype(jnp.int32)
    segB_i32 = herb_batch_B.astype(jnp.int32)
    segA = segA_i32.reshape(NB, 1, R)
    segB = segB_i32.reshape(NB, 1, R)
    smin = jnp.stack([segA[:, 0, 0], segB[:, 0, 0]])
    smax = jnp.stack([segA[:, 0, R - 1], segB[:, 0, R - 1]])

    sums = _sc_pass1(x_A, segA_i32, x_B, segB_i32)

    G = pl.pallas_call(
        _g_body,
        out_shape=jax.ShapeDtypeStruct((2, S, D), jnp.float32),
        interpret=_INTERPRET,
    )(segA, segB, sums, W_gnn, W_attn)

    xspec = pl.BlockSpec((R, D), lambda i, a, b: (i, 0))
    sspec2 = pl.BlockSpec((1, 1, R), lambda i, a, b: (i, 0, 0))
    full2p = pl.BlockSpec((2, S, D), lambda i, a, b: (0, 0, 0))
    full1p = pl.BlockSpec((2, S), lambda i, a, b: (0, 0))

    grid_spec = pltpu.PrefetchScalarGridSpec(
        num_scalar_prefetch=2,
        grid=(NB,),
        in_specs=[xspec, sspec2, xspec, sspec2, full2p],
        out_specs=[full2p, full1p, full1p],
        scratch_shapes=[pltpu.VMEM((R, D), jnp.float32)],
    )
    U, m, d = pl.pallas_call(
        _pass2_body,
        grid_spec=grid_spec,
        out_shape=[jax.ShapeDtypeStruct((2, S, D), jnp.float32),
                   jax.ShapeDtypeStruct((2, S), jnp.float32),
                   jax.ShapeDtypeStruct((2, S), jnp.float32)],
        compiler_params=pltpu.CompilerParams(
            dimension_semantics=("arbitrary",)),
        interpret=_INTERPRET,
    )(smin, smax, x_A, segA, x_B, segB, G)

    H_A, H_B = pl.pallas_call(
        _final_body,
        out_shape=[jax.ShapeDtypeStruct((S, D), jnp.float32),
                   jax.ShapeDtypeStruct((S, D), jnp.float32)],
        interpret=_INTERPRET,
    )(U, d, W_gnn)
    return (H_A, H_B)


# speedup vs baseline: 1.0871x; 1.0871x over previous
import functools

import jax
import jax.numpy as jnp
from jax import lax
from jax.experimental import pallas as pl
from jax.experimental.pallas import tpu as pltpu
from jax.experimental.pallas import tpu_sc as plsc

N = 100000
D = 128
S = 256
R = 2000
NB = N // R
NEG = -1e30
SW = 64
NST = S // SW

NC, NS, NW = 2, 16, 32
T = 400
SB = 80
NSB = T // SB
NCHUNK = N // T
CPW = (NCHUNK + NW - 1) // NW

_INTERPRET = False


def _onehot(seg, dtype=jnp.float32):
    cols = lax.broadcasted_iota(jnp.int32, (R, S), 1)
    return jnp.where(seg[:, None] == cols, jnp.array(1.0, dtype), jnp.array(0.0, dtype))


def _sc_pass1_body(xa_hbm, sa_hbm, xb_hbm, sb_hbm, sums_out,
                   xbufs, idxbufs, zbuf, semx0, semx1,
                   tab_a, tab_b):
    cid = lax.axis_index("c")
    sid = lax.axis_index("s")
    w = sid * NC + cid
    sems = (semx0, semx1)

    zero = jnp.zeros((16,), jnp.float32)
    for r in range(16):
        for j in range(D // 16):
            zbuf[r, pl.ds(16 * j, 16)] = zero

    row0 = sid * 16
    pltpu.sync_copy(zbuf, tab_a.at[pl.ds(row0, 16)])
    pltpu.sync_copy(zbuf, tab_b.at[pl.ds(row0, 16)])
    plsc.subcore_barrier()

    def _copies(x_hbm, s_hbm, k, b):
        off = k * T
        sem = sems[b]
        cps = [pltpu.make_async_copy(x_hbm.at[pl.ds(off, T)], xbufs.at[b], sem)]
        for i in range(NSB):
            cps.append(pltpu.make_async_copy(
                s_hbm.at[pl.ds(off + i * SB, SB)], idxbufs.at[b, i], sem))
        return cps

    for x_hbm, s_hbm, tab in ((xa_hbm, sa_hbm, tab_a),
                              (xb_hbm, sb_hbm, tab_b)):
        @pl.when(w < NCHUNK)
        def _():
            for cp in _copies(x_hbm, s_hbm, w, 0):
                cp.start()

        for j in range(CPW):
            k = w + NW * j
            b = j % 2
            if j + 1 < CPW:
                k_next = k + NW

                @pl.when(k_next < NCHUNK)
                def _():
                    for cp in _copies(x_hbm, s_hbm, k_next, 1 - b):
                        cp.start()

            @pl.when(k < NCHUNK)
            def _():
                for cp in _copies(x_hbm, s_hbm, k, b):
                    cp.wait()
                for i in range(NSB):
                    pltpu.sync_copy(xbufs.at[b, pl.ds(i * SB, SB)],
                                    tab.at[idxbufs.at[b, i]], add=True)

    plsc.subcore_barrier()

    @pl.when(sid == 0)
    def _():
        pltpu.sync_copy(tab_a, sums_out.at[0, cid])
        pltpu.sync_copy(tab_b, sums_out.at[1, cid])


def _sc_pass1(x_A, segA_i32, x_B, segB_i32):
    return pl.kernel(
        _sc_pass1_body,
        out_type=jax.ShapeDtypeStruct((2, NC, S, D), jnp.float32),
        mesh=plsc.VectorSubcoreMesh(core_axis_name="c", subcore_axis_name="s"),
        scratch_types=[
            pltpu.VMEM((2, T, D), jnp.float32),
            pltpu.VMEM((2, NSB, SB), jnp.int32),
            pltpu.VMEM((16, D), jnp.float32),
            pltpu.SemaphoreType.DMA,
            pltpu.SemaphoreType.DMA,
            pltpu.VMEM_SHARED((S, D), jnp.float32),
            pltpu.VMEM_SHARED((S, D), jnp.float32),
        ],
    )(x_A, segA_i32, x_B, segB_i32)


def _cnt_body(smin_ref, smax_ref, sa_ref, sb_ref, cnt_ref):
    cnt_ref[...] = jnp.zeros_like(cnt_ref)
    H = S // 2
    ones = jnp.ones((R, 1), jnp.float32)
    for side, s_ref in enumerate((sa_ref, sb_ref)):
        for j in range(NB):
            seg = s_ref[j, 0, :]
            smin = smin_ref[side, j]
            smax = smax_ref[side, j]

            def _win(ws, wh):
                ohb = (seg - ws)[:, None] == lax.broadcasted_iota(
                    jnp.int32, (R, wh), 1)
                oh = jnp.where(ohb, 1.0, 0.0)
                dd = lax.dot_general(oh, ones, (((0,), (0,)), ((), ())),
                                     preferred_element_type=jnp.float32)
                cnt_ref[side, pl.ds(ws, wh)] += dd[:, 0]

            @pl.when(smax < H)
            def _():
                _win(0, H)

            @pl.when(smin >= H)
            def _():
                _win(H, H)

            @pl.when((smax >= H) & (smin < H))
            def _():
                _win(0, S)


def _g_body(sums_ref, cnt_ref, wg_ref, wa_ref, g_ref):
    wg = wg_ref[...]
    wa = wa_ref[...]
    m1 = jnp.dot(wg, wa, preferred_element_type=jnp.float32)
    M = jnp.dot(m1, wg.T, preferred_element_type=jnp.float32)
    sums = sums_ref[...]
    tot = sums[:, 0] + sums[:, 1]
    cnt = jnp.maximum(cnt_ref[...], 1.0)
    mean = tot / cnt[:, :, None]
    g_ref[0] = jnp.dot(mean[1], M, preferred_element_type=jnp.float32)
    g_ref[1] = jnp.dot(mean[0], M, preferred_element_type=jnp.float32)


def _pass2_window(side, ws, wh, x, seg, g_ref, u_ref, m_ref, d_ref):
    segl = seg - ws
    ohb = segl[:, None] == lax.broadcasted_iota(jnp.int32, (R, wh), 1)
    oh = jnp.where(ohb, 1.0, 0.0)
    gmat = lax.dot_general(oh, g_ref[side, pl.ds(ws, wh), :],
                           (((1,), (0,)), ((), ())),
                           preferred_element_type=jnp.float32)
    e = jnp.sum(x * gmat, axis=1)
    me = jnp.max(jnp.where(ohb, e[:, None], NEG), axis=0)
    m_old = m_ref[side, pl.ds(ws, wh)]
    m_new = jnp.maximum(m_old, me)
    m_gath = lax.dot_general(oh, m_new[:, None], (((1,), (0,)), ((), ())),
                             preferred_element_type=jnp.float32)
    w = jnp.exp(e[:, None] - m_gath)
    scale = jnp.exp(m_old - m_new)
    dd = lax.dot_general(oh, w, (((0,), (0,)), ((), ())),
                         preferred_element_type=jnp.float32)
    d_ref[side, pl.ds(ws, wh)] = d_ref[side, pl.ds(ws, wh)] * scale + dd[:, 0]
    wx = x * w
    u_ref[side, pl.ds(ws, wh), :] = (
        u_ref[side, pl.ds(ws, wh), :] * scale[:, None]
        + lax.dot_general(oh, wx, (((0,), (0,)), ((), ())),
                          preferred_element_type=jnp.float32))
    m_ref[side, pl.ds(ws, wh)] = m_new


def _pass2_body(smin_ref, smax_ref, xa_ref, sa_ref, xb_ref, sb_ref, g_ref,
                u_ref, m_ref, d_ref):
    i = pl.program_id(0)

    @pl.when(i == 0)
    def _():
        u_ref[...] = jnp.zeros_like(u_ref)
        m_ref[...] = jnp.full_like(m_ref, NEG)
        d_ref[...] = jnp.zeros_like(d_ref)

    H = S // 2
    for side, (x_ref, s_ref) in enumerate(((xa_ref, sa_ref), (xb_ref, sb_ref))):
        seg = s_ref[0, 0, :]
        smin = smin_ref[side, i]
        smax = smax_ref[side, i]
        x = x_ref[...]

        @pl.when(smax < H)
        def _():
            _pass2_window(side, 0, H, x, seg, g_ref, u_ref, m_ref, d_ref)

        @pl.when(smin >= H)
        def _():
            _pass2_window(side, H, H, x, seg, g_ref, u_ref, m_ref, d_ref)

        @pl.when((smax >= H) & (smin < H))
        def _():
            _pass2_window(side, 0, S, x, seg, g_ref, u_ref, m_ref, d_ref)


def _final_body(u_ref, d_ref, wg_ref, outa_ref, outb_ref):
    wg = wg_ref[...]
    den = d_ref[...] + 1e-16
    pooled = u_ref[...] / den[:, :, None]
    outa_ref[...] = jnp.dot(pooled[0], wg, preferred_element_type=jnp.float32)
    outb_ref[...] = jnp.dot(pooled[1], wg, preferred_element_type=jnp.float32)


def kernel(x_A, herb_batch_A, x_B, herb_batch_B, W_gnn, W_attn):
    segA_i32 = herb_batch_A.astype(jnp.int32)
    segB_i32 = herb_batch_B.astype(jnp.int32)
    segA = segA_i32.reshape(NB, 1, R)
    segB = segB_i32.reshape(NB, 1, R)
    smin = jnp.stack([segA[:, 0, 0], segB[:, 0, 0]])
    smax = jnp.stack([segA[:, 0, R - 1], segB[:, 0, R - 1]])

    cnt_grid = pltpu.PrefetchScalarGridSpec(
        num_scalar_prefetch=2,
        grid=(1,),
        in_specs=[pl.BlockSpec((NB, 1, R), lambda i, a, b: (0, 0, 0)),
                  pl.BlockSpec((NB, 1, R), lambda i, a, b: (0, 0, 0))],
        out_specs=pl.BlockSpec((2, S), lambda i, a, b: (0, 0)),
    )
    cnt = pl.pallas_call(
        _cnt_body,
        grid_spec=cnt_grid,
        out_shape=jax.ShapeDtypeStruct((2, S), jnp.float32),
        interpret=_INTERPRET,
    )(smin, smax, segA, segB)

    sums = _sc_pass1(x_A, segA_i32, x_B, segB_i32)

    G = pl.pallas_call(
        _g_body,
        out_shape=jax.ShapeDtypeStruct((2, S, D), jnp.float32),
        interpret=_INTERPRET,
    )(sums, cnt, W_gnn, W_attn)

    xspec = pl.BlockSpec((R, D), lambda i, a, b: (i, 0))
    sspec2 = pl.BlockSpec((1, 1, R), lambda i, a, b: (i, 0, 0))
    full2p = pl.BlockSpec((2, S, D), lambda i, a, b: (0, 0, 0))
    full1p = pl.BlockSpec((2, S), lambda i, a, b: (0, 0))

    grid_spec = pltpu.PrefetchScalarGridSpec(
        num_scalar_prefetch=2,
        grid=(NB,),
        in_specs=[xspec, sspec2, xspec, sspec2, full2p],
        out_specs=[full2p, full1p, full1p],
    )
    U, m, d = pl.pallas_call(
        _pass2_body,
        grid_spec=grid_spec,
        out_shape=[jax.ShapeDtypeStruct((2, S, D), jnp.float32),
                   jax.ShapeDtypeStruct((2, S), jnp.float32),
                   jax.ShapeDtypeStruct((2, S), jnp.float32)],
        compiler_params=pltpu.CompilerParams(
            dimension_semantics=("arbitrary",)),
        interpret=_INTERPRET,
    )(smin, smax, x_A, segA, x_B, segB, G)

    H_A, H_B = pl.pallas_call(
        _final_body,
        out_shape=[jax.ShapeDtypeStruct((S, D), jnp.float32),
                   jax.ShapeDtypeStruct((S, D), jnp.float32)],
        interpret=_INTERPRET,
    )(U, d, W_gnn)
    return (H_A, H_B)
